# baseline (speedup 1.0000x reference)
import functools

import jax
import jax.numpy as jnp
from jax import lax
from jax.experimental import pallas as pl
from jax.experimental.pallas import tpu as pltpu
from jax.experimental.pallas import tpu_sc as plsc

B, T, H, D = 1024, 200, 8, 32
N = B * T * H

NC, NS, L = 2, 16, 16
NW = NC * NS
PER_W = N // NW
C = 128
S = PER_W // C
K = 8
OUTER = S // K


def _sc_gather(idx_hbm, table_hbm, off_hbm):
    mesh = plsc.VectorSubcoreMesh(core_axis_name="c", subcore_axis_name="s")

    @functools.partial(
        pl.kernel,
        out_type=jax.ShapeDtypeStruct((N, 128), jnp.float32),
        mesh=mesh,
        compiler_params=pltpu.CompilerParams(use_tc_tiling_on_sc=False),
        scratch_types=[
            pltpu.VMEM((S, C), jnp.int32),
            pltpu.VMEM((16,), jnp.int32),
            pltpu.VMEM((K * C, D), jnp.float32),
            pltpu.SemaphoreType.DMA,
        ],
    )
    def k(idx_ref, table_ref, off_ref, out_ref, idx_v, off_v, rows_v, gsem):
        wid = lax.axis_index("s") * NC + lax.axis_index("c")
        base = wid * PER_W

        pltpu.sync_copy(off_ref, off_v.at[pl.ds(0, H)])
        pltpu.sync_copy(off_ref, off_v.at[pl.ds(H, H)])
        pltpu.sync_copy(idx_ref.at[pl.ds(wid * S, S)], idx_v)
        off = off_v[...]

        def add_body(s, carry):
            for i in range(C // L):
                sl = pl.ds(i * L, L)
                idx_v[s, sl] = idx_v[s, sl] + off
            return carry

        lax.fori_loop(0, S, add_body, 0)

        def outer_body(j, carry):
            copies = []
            for bk in range(K):
                cp = pltpu.async_copy(
                    table_ref.at[idx_v.at[j * K + bk]],
                    rows_v.at[pl.ds(bk * C, C)],
                    gsem,
                )
                copies.append(cp)
            for cp in copies:
                cp.wait()
            pltpu.sync_copy(rows_v,
                            out_ref.at[pl.ds(base + j * (K * C), K * C),
                                       pl.ds(0, D)])
            return carry

        lax.fori_loop(0, OUTER, outer_body, 0)

    return k(idx_hbm, table_hbm, off_hbm)


def kernel(input_ids, table, offsets):
    out = _sc_gather(input_ids.reshape(N // C, C), table, offsets)
    return out[:, :D].reshape(B, T, H, D)


# speedup vs baseline: 1.0090x; 1.0090x over previous
import functools

import jax
import jax.numpy as jnp
from jax import lax
from jax.experimental import pallas as pl
from jax.experimental.pallas import tpu as pltpu
from jax.experimental.pallas import tpu_sc as plsc

B, T, H, D = 1024, 200, 8, 32
N = B * T * H

NC, NS, L = 2, 16, 16
NW = NC * NS
PER_W = N // NW
C = 128
S = PER_W // C
K = 5
BLKS = S // K


def _sc_gather(idx_hbm, table_hbm, off_hbm):
    mesh = plsc.VectorSubcoreMesh(core_axis_name="c", subcore_axis_name="s")

    @functools.partial(
        pl.kernel,
        out_type=jax.ShapeDtypeStruct((N, 128), jnp.float32),
        mesh=mesh,
        compiler_params=pltpu.CompilerParams(use_tc_tiling_on_sc=False),
        scratch_types=[
            pltpu.VMEM((S, C), jnp.int32),
            pltpu.VMEM((16,), jnp.int32),
            pltpu.VMEM((K * C, D), jnp.float32),
            pltpu.VMEM((K * C, D), jnp.float32),
            pltpu.SemaphoreType.DMA,
            pltpu.SemaphoreType.DMA,
            pltpu.SemaphoreType.DMA,
        ],
    )
    def k(idx_ref, table_ref, off_ref, out_ref, idx_v, off_v, r0, r1,
          gsem, w0, w1):
        wid = lax.axis_index("s") * NC + lax.axis_index("c")
        base = wid * PER_W

        pltpu.sync_copy(off_ref, off_v.at[pl.ds(0, H)])
        pltpu.sync_copy(off_ref, off_v.at[pl.ds(H, H)])
        pltpu.sync_copy(idx_ref.at[pl.ds(wid * S, S)], idx_v)
        off = off_v[...]

        def add_body(s, carry):
            for i in range(C // L):
                sl = pl.ds(i * L, L)
                idx_v[s, sl] = idx_v[s, sl] + off
            return carry

        lax.fori_loop(0, S, add_body, 0)

        def outer_body(jo, carry):
            for bi, (rb, wsem) in enumerate(((r0, w0), (r1, w1))):
                j = 2 * jo + bi

                @pl.when(jo > 0)
                def _wait_prev():
                    pltpu.make_async_copy(
                        rb, out_ref.at[pl.ds(base + j * (K * C), K * C),
                                       pl.ds(0, D)], wsem).wait()

                copies = []
                for g in range(K):
                    copies.append(pltpu.async_copy(
                        table_ref.at[idx_v.at[j * K + g]],
                        rb.at[pl.ds(g * C, C)], gsem))
                for cp in copies:
                    cp.wait()
                pltpu.async_copy(
                    rb, out_ref.at[pl.ds(base + j * (K * C), K * C),
                                   pl.ds(0, D)], wsem)
            return carry

        lax.fori_loop(0, BLKS // 2, outer_body, 0)
        for rb, wsem in ((r0, w0), (r1, w1)):
            pltpu.make_async_copy(
                rb, out_ref.at[pl.ds(base, K * C), pl.ds(0, D)], wsem).wait()

    return k(idx_hbm, table_hbm, off_hbm)


def kernel(input_ids, table, offsets):
    out = _sc_gather(input_ids.reshape(N // C, C), table, offsets)
    return out[:, :D].reshape(B, T, H, D)


# speedup vs baseline: 1.0334x; 1.0242x over previous
import functools

import jax
import jax.numpy as jnp
from jax import lax
from jax.experimental import pallas as pl
from jax.experimental.pallas import tpu as pltpu
from jax.experimental.pallas import tpu_sc as plsc

B, T, H, D = 1024, 200, 8, 32
N = B * T * H

NC, NS, L = 2, 16, 16
NW = NC * NS
PER_W = N // NW
C = 128
S = PER_W // C
K = 8
BLKS = S // K


def _sc_gather(idx_hbm, table_hbm, off_hbm):
    mesh = plsc.VectorSubcoreMesh(core_axis_name="c", subcore_axis_name="s")

    @functools.partial(
        pl.kernel,
        out_type=jax.ShapeDtypeStruct((N, 128), jnp.float32),
        mesh=mesh,
        compiler_params=pltpu.CompilerParams(use_tc_tiling_on_sc=False),
        scratch_types=[
            pltpu.VMEM((S, C), jnp.int32),
            pltpu.VMEM((16,), jnp.int32),
            pltpu.VMEM((K * C, D), jnp.float32),
            pltpu.VMEM((K * C, D), jnp.float32),
            pltpu.SemaphoreType.DMA,
            pltpu.SemaphoreType.DMA,
            pltpu.SemaphoreType.DMA,
        ],
    )
    def k(idx_ref, table_ref, off_ref, out_ref, idx_v, off_v, r0, r1,
          gsem, w0, w1):
        wid = lax.axis_index("s") * NC + lax.axis_index("c")
        base = wid * PER_W

        pltpu.sync_copy(off_ref, off_v.at[pl.ds(0, H)])
        pltpu.sync_copy(off_ref, off_v.at[pl.ds(H, H)])
        pltpu.sync_copy(idx_ref.at[pl.ds(wid * S, S)], idx_v)
        off = off_v[...]

        def add_body(s, carry):
            for i in range(C // L):
                sl = pl.ds(i * L, L)
                idx_v[s, sl] = idx_v[s, sl] + off
            return carry

        lax.fori_loop(0, S, add_body, 0)

        def outer_body(jo, carry):
            for bi, (rb, wsem) in enumerate(((r0, w0), (r1, w1))):
                j = 2 * jo + bi

                @pl.when(jo > 0)
                def _wait_prev():
                    pltpu.make_async_copy(
                        rb, out_ref.at[pl.ds(base + j * (K * C), K * C),
                                       pl.ds(0, D)], wsem).wait()

                copies = []
                for g in range(K):
                    copies.append(pltpu.async_copy(
                        table_ref.at[idx_v.at[j * K + g]],
                        rb.at[pl.ds(g * C, C)], gsem))
                for cp in copies:
                    cp.wait()
                pltpu.async_copy(
                    rb, out_ref.at[pl.ds(base + j * (K * C), K * C),
                                   pl.ds(0, D)], wsem)
            return carry

        lax.fori_loop(0, BLKS // 2, outer_body, 0)
        for rb, wsem in ((r0, w0), (r1, w1)):
            pltpu.make_async_copy(
                rb, out_ref.at[pl.ds(base, K * C), pl.ds(0, D)], wsem).wait()

    return k(idx_hbm, table_hbm, off_hbm)


def kernel(input_ids, table, offsets):
    out = _sc_gather(input_ids.reshape(N // C, C), table, offsets)
    return out[:, :D].reshape(B, T, H, D)


# speedup vs baseline: 1.0336x; 1.0002x over previous
import functools

import jax
import jax.numpy as jnp
from jax import lax
from jax.experimental import pallas as pl
from jax.experimental.pallas import tpu as pltpu
from jax.experimental.pallas import tpu_sc as plsc

B, T, H, D = 1024, 200, 8, 32
N = B * T * H

NC, NS, L = 2, 16, 16
NW = NC * NS
PER_W = N // NW
C = 128
S = PER_W // C
K = 8
BLKS = S // K


def _sc_gather(idx_hbm, table_hbm, off_hbm):
    mesh = plsc.VectorSubcoreMesh(core_axis_name="c", subcore_axis_name="s")

    @functools.partial(
        pl.kernel,
        out_type=jax.ShapeDtypeStruct((N, 128), jnp.float32),
        mesh=mesh,
        compiler_params=pltpu.CompilerParams(use_tc_tiling_on_sc=False),
        scratch_types=[
            pltpu.VMEM((S, C), jnp.int32),
            pltpu.VMEM((16,), jnp.int32),
            pltpu.VMEM((K * C, D), jnp.float32),
            pltpu.VMEM((K * C, D), jnp.float32),
            pltpu.SemaphoreType.DMA,
            pltpu.SemaphoreType.DMA,
            pltpu.SemaphoreType.DMA,
        ],
    )
    def k(idx_ref, table_ref, off_ref, out_ref, idx_v, off_v, r0, r1,
          gsem, w0, w1):
        wid = lax.axis_index("s") * NC + lax.axis_index("c")
        base = wid * PER_W

        pltpu.sync_copy(off_ref, off_v.at[pl.ds(0, H)])
        pltpu.sync_copy(off_ref, off_v.at[pl.ds(H, H)])
        pltpu.sync_copy(idx_ref.at[pl.ds(wid * S, S)], idx_v)
        off = off_v[...]

        def add_rows(row0):
            for rr in range(K):
                for i in range(C // L):
                    sl = pl.ds(i * L, L)
                    idx_v[row0 + rr, sl] = idx_v[row0 + rr, sl] + off

        for jj in range(2):
            add_rows(jj * K)

        def outer_body(jo, carry):
            for bi, (rb, wsem) in enumerate(((r0, w0), (r1, w1))):
                j = 2 * jo + bi

                @pl.when(jo > 0)
                def _wait_prev():
                    pltpu.make_async_copy(
                        rb, out_ref.at[pl.ds(base + j * (K * C), K * C),
                                       pl.ds(0, D)], wsem).wait()

                copies = []
                for g in range(K):
                    copies.append(pltpu.async_copy(
                        table_ref.at[idx_v.at[j * K + g]],
                        rb.at[pl.ds(g * C, C)], gsem))

                @pl.when(j < BLKS - 2)
                def _shift_ahead():
                    add_rows((j + 2) * K)

                for cp in copies:
                    cp.wait()
                pltpu.async_copy(
                    rb, out_ref.at[pl.ds(base + j * (K * C), K * C),
                                   pl.ds(0, D)], wsem)
            return carry

        lax.fori_loop(0, BLKS // 2, outer_body, 0)
        for rb, wsem in ((r0, w0), (r1, w1)):
            pltpu.make_async_copy(
                rb, out_ref.at[pl.ds(base, K * C), pl.ds(0, D)], wsem).wait()

    return k(idx_hbm, table_hbm, off_hbm)


def kernel(input_ids, table, offsets):
    out = _sc_gather(input_ids.reshape(N // C, C), table, offsets)
    return out[:, :D].reshape(B, T, H, D)
